# baseline (speedup 1.0000x reference)
import functools

import jax
import jax.numpy as jnp
from jax import lax
from jax.experimental import pallas as pl
from jax.experimental.pallas import tpu as pltpu
from jax.experimental.pallas import tpu_sc as plsc

N_NODES = 10000
N_EDGES = 320000
D = 128

NC = 2
NS = 16
NW = NC * NS
E_PER_W = N_EDGES // NW
K = 80
CHUNKS = E_PER_W // K
NP = 10240
ROWS_PER_TILE = NP // NS
SHIFT = 14
FBLK = 1280


def _pack(ei):
    def body(e_ref, o_ref):
        o_ref[...] = (e_ref[1] << SHIFT) | e_ref[0]

    return pl.pallas_call(
        body,
        out_shape=jax.ShapeDtypeStruct((N_EDGES // D, D), jnp.int32),
    )(ei.reshape(2, N_EDGES // D, D))


def _finalize(partials, pdeg, Wt, b2):
    def body(p_ref, d_ref, w_ref, b_ref, o_ref):
        s = p_ref[0] + p_ref[1]
        deg = d_ref[0] + d_ref[1]
        clip = jnp.maximum(deg, 1.0)
        sn = s / clip[:, None]
        scale = (deg / clip)[:, None]
        o_ref[...] = (
            jnp.dot(sn, w_ref[...], preferred_element_type=jnp.float32)
            + scale * b_ref[...]
        )

    return pl.pallas_call(
        body,
        grid=(NP // FBLK,),
        in_specs=[
            pl.BlockSpec((NC, FBLK, D), lambda i: (0, i, 0)),
            pl.BlockSpec((NC, FBLK), lambda i: (0, i)),
            pl.BlockSpec((D, D), lambda i: (0, 0)),
            pl.BlockSpec((1, D), lambda i: (0, 0)),
        ],
        out_specs=pl.BlockSpec((FBLK, D), lambda i: (i, 0)),
        out_shape=jax.ShapeDtypeStruct((N_NODES, D), jnp.float32),
    )(partials, pdeg, Wt, b2)


def _sc_aggregate(x, packed):
    mesh = plsc.VectorSubcoreMesh(core_axis_name="c", subcore_axis_name="s")

    @functools.partial(
        pl.kernel,
        mesh=mesh,
        out_type=[
            jax.ShapeDtypeStruct((NC, NP, D), jnp.float32),
            jax.ShapeDtypeStruct((NC, NP), jnp.float32),
        ],
        scratch_types=[
            pltpu.VMEM((E_PER_W,), jnp.int32),
            pltpu.VMEM((K,), jnp.int32),
            pltpu.VMEM((K,), jnp.int32),
            pltpu.VMEM((K,), jnp.int32),
            pltpu.VMEM((K,), jnp.int32),
            pltpu.VMEM((K,), jnp.int32),
            pltpu.VMEM((K,), jnp.int32),
            pltpu.VMEM((K, D), jnp.float32),
            pltpu.VMEM((K, D), jnp.float32),
            pltpu.VMEM((K, D), jnp.float32),
            pltpu.VMEM((K,), jnp.float32),
            pltpu.VMEM((ROWS_PER_TILE,), jnp.float32),
            pltpu.VMEM_SHARED((NP, D), jnp.float32),
            pltpu.VMEM_SHARED((NP,), jnp.float32),
            pltpu.SemaphoreType.DMA,
            pltpu.SemaphoreType.DMA,
            pltpu.SemaphoreType.DMA,
            pltpu.SemaphoreType.DMA,
        ],
    )
    def k(x_hbm, packed_hbm, part_hbm, pdeg_hbm,
          packed_v, srcb_a, srcb_b, srcb_c, dstb_a, dstb_b, dstb_c,
          rows_a, rows_b, rows_c, ones_v,
          zdeg_v, acc_sh, deg_sh, sem_a, sem_b, sem_c, sem_d):
        cid = lax.axis_index("c")
        tid = lax.axis_index("s")
        wid = cid * NS + tid
        r0 = tid * ROWS_PER_TILE

        pltpu.sync_copy(
            packed_hbm.at[pl.ds(wid * E_PER_W, E_PER_W)], packed_v)

        z16 = jnp.zeros((16,), jnp.float32)

        def zrow(r, c):
            for j in range(8):
                rows_a[r, pl.ds(j * 16, 16)] = z16
            return c

        lax.fori_loop(0, K, zrow, 0)

        def zdeg(i, c):
            zdeg_v[pl.ds(i * 16, 16)] = z16
            return c

        lax.fori_loop(0, ROWS_PER_TILE // 16, zdeg, 0)

        for j in range(ROWS_PER_TILE // K):
            pltpu.async_copy(rows_a, acc_sh.at[pl.ds(r0 + j * K, K)], sem_a)
        pltpu.sync_copy(zdeg_v, deg_sh.at[pl.ds(r0, ROWS_PER_TILE)])
        for j in range(ROWS_PER_TILE // K):
            pltpu.make_async_copy(
                rows_a, acc_sh.at[pl.ds(r0, K)], sem_a).wait()

        for i in range(K // 16):
            ones_v[pl.ds(i * 16, 16)] = jnp.full((16,), 1.0, jnp.float32)

        mask = jnp.full((16,), (1 << SHIFT) - 1, jnp.int32)

        def unpack(t, srcb, dstb):
            for kk in range(K // 16):
                v = packed_v[pl.ds(t * K + kk * 16, 16)]
                srcb[pl.ds(kk * 16, 16)] = v & mask
                dstb[pl.ds(kk * 16, 16)] = lax.shift_right_logical(v, SHIFT)

        def gather(srcb, rows, sem):
            pltpu.async_copy(x_hbm.at[srcb], rows, sem)

        def gather_wait(rows, sem):
            pltpu.make_async_copy(x_hbm.at[srcb_a], rows, sem).wait()

        def deg_start(dstb):
            pltpu.async_copy(ones_v, deg_sh.at[dstb], sem_d, add=True)

        def deg_wait(dstb):
            pltpu.make_async_copy(ones_v, deg_sh.at[dstb], sem_d).wait()

        def scatter(dstb, rows):
            pltpu.sync_copy(rows, acc_sh.at[dstb], add=True)

        unpack(0, srcb_a, dstb_a)
        unpack(1, srcb_b, dstb_b)
        unpack(2, srcb_c, dstb_c)

        plsc.subcore_barrier()

        gather(srcb_a, rows_a, sem_a)
        gather(srcb_b, rows_b, sem_b)
        gather(srcb_c, rows_c, sem_c)

        def chunk_step(t3, srcb, dstb, rows, sem):
            gather_wait(rows, sem)
            deg_start(dstb)
            scatter(dstb, rows)
            deg_wait(dstb)

            @pl.when(t3 < CHUNKS)
            def _():
                unpack(t3, srcb, dstb)
                gather(srcb, rows, sem)

        def triple_body(i, c):
            t = 3 * i
            chunk_step(t + 3, srcb_a, dstb_a, rows_a, sem_a)
            chunk_step(t + 4, srcb_b, dstb_b, rows_b, sem_b)
            chunk_step(t + 5, srcb_c, dstb_c, rows_c, sem_c)
            return c

        lax.fori_loop(0, CHUNKS // 3, triple_body, 0)

        chunk_step(CHUNKS + 1, srcb_a, dstb_a, rows_a, sem_a)
        chunk_step(CHUNKS + 2, srcb_b, dstb_b, rows_b, sem_b)

        plsc.subcore_barrier()

        pltpu.sync_copy(
            acc_sh.at[pl.ds(r0, ROWS_PER_TILE)],
            part_hbm.at[cid, pl.ds(r0, ROWS_PER_TILE)],
        )

        @pl.when(tid == 0)
        def _():
            pltpu.sync_copy(deg_sh, pdeg_hbm.at[cid])

    return k(x, packed)


def kernel(x, edge_index, W, b):
    ei = edge_index.astype(jnp.int32)
    packed = _pack(ei).reshape(N_EDGES)

    partials, pdeg = _sc_aggregate(x, packed)
    return _finalize(partials, pdeg, W.T, b.reshape(1, D))


# speedup vs baseline: 1.0153x; 1.0153x over previous
import functools

import jax
import jax.numpy as jnp
from jax import lax
from jax.experimental import pallas as pl
from jax.experimental.pallas import tpu as pltpu
from jax.experimental.pallas import tpu_sc as plsc

N_NODES = 10000
N_EDGES = 320000
D = 128

NC = 2
NS = 16
NW = NC * NS
E_PER_W = N_EDGES // NW
K = 40
CHUNKS = E_PER_W // K
KOFF = (0, 16, 24)
NP = 10240
ROWS_PER_TILE = NP // NS
SHIFT = 14
FBLK = 1280


def _pack(ei):
    def body(e_ref, o_ref):
        o_ref[...] = (e_ref[1] << SHIFT) | e_ref[0]

    return pl.pallas_call(
        body,
        out_shape=jax.ShapeDtypeStruct((N_EDGES // D, D), jnp.int32),
    )(ei.reshape(2, N_EDGES // D, D))


def _finalize(partials, pdeg, Wt, b2):
    def body(p_ref, d_ref, w_ref, b_ref, o_ref):
        s = p_ref[0] + p_ref[1]
        deg = d_ref[0] + d_ref[1]
        clip = jnp.maximum(deg, 1.0)
        sn = s / clip[:, None]
        scale = (deg / clip)[:, None]
        o_ref[...] = (
            jnp.dot(sn, w_ref[...], preferred_element_type=jnp.float32)
            + scale * b_ref[...]
        )

    return pl.pallas_call(
        body,
        grid=(NP // FBLK,),
        in_specs=[
            pl.BlockSpec((NC, FBLK, D), lambda i: (0, i, 0)),
            pl.BlockSpec((NC, FBLK), lambda i: (0, i)),
            pl.BlockSpec((D, D), lambda i: (0, 0)),
            pl.BlockSpec((1, D), lambda i: (0, 0)),
        ],
        out_specs=pl.BlockSpec((FBLK, D), lambda i: (i, 0)),
        out_shape=jax.ShapeDtypeStruct((N_NODES, D), jnp.float32),
    )(partials, pdeg, Wt, b2)


def _sc_aggregate(x, packed):
    mesh = plsc.VectorSubcoreMesh(core_axis_name="c", subcore_axis_name="s")

    @functools.partial(
        pl.kernel,
        mesh=mesh,
        out_type=[
            jax.ShapeDtypeStruct((NC, NP, D), jnp.float32),
            jax.ShapeDtypeStruct((NC, NP), jnp.float32),
        ],
        scratch_types=[
            pltpu.VMEM((E_PER_W,), jnp.int32),
            pltpu.VMEM((K,), jnp.int32),
            pltpu.VMEM((K,), jnp.int32),
            pltpu.VMEM((K,), jnp.int32),
            pltpu.VMEM((K,), jnp.int32),
            pltpu.VMEM((K,), jnp.int32),
            pltpu.VMEM((K,), jnp.int32),
            pltpu.VMEM((K,), jnp.int32),
            pltpu.VMEM((K,), jnp.int32),
            pltpu.VMEM((K,), jnp.int32),
            pltpu.VMEM((K,), jnp.int32),
            pltpu.VMEM((K, D), jnp.float32),
            pltpu.VMEM((K, D), jnp.float32),
            pltpu.VMEM((K, D), jnp.float32),
            pltpu.VMEM((K, D), jnp.float32),
            pltpu.VMEM((K, D), jnp.float32),
            pltpu.VMEM((K,), jnp.float32),
            pltpu.VMEM((ROWS_PER_TILE,), jnp.float32),
            pltpu.VMEM_SHARED((NP, D), jnp.float32),
            pltpu.VMEM_SHARED((NP,), jnp.float32),
            pltpu.SemaphoreType.DMA,
            pltpu.SemaphoreType.DMA,
            pltpu.SemaphoreType.DMA,
            pltpu.SemaphoreType.DMA,
            pltpu.SemaphoreType.DMA,
            pltpu.SemaphoreType.DMA,
        ],
    )
    def k(x_hbm, packed_hbm, part_hbm, pdeg_hbm,
          packed_v, srcb_a, srcb_b, srcb_c, srcb_d, srcb_e,
          dstb_a, dstb_b, dstb_c, dstb_d, dstb_e,
          rows_a, rows_b, rows_c, rows_d, rows_e, ones_v,
          zdeg_v, acc_sh, deg_sh,
          sem_a, sem_b, sem_c, sem_d2, sem_e, sem_d):
        cid = lax.axis_index("c")
        tid = lax.axis_index("s")
        wid = cid * NS + tid
        r0 = tid * ROWS_PER_TILE

        pltpu.sync_copy(
            packed_hbm.at[pl.ds(wid * E_PER_W, E_PER_W)], packed_v)

        z16 = jnp.zeros((16,), jnp.float32)

        def zrow(r, c):
            for j in range(8):
                rows_a[r, pl.ds(j * 16, 16)] = z16
            return c

        lax.fori_loop(0, K, zrow, 0)

        def zdeg(i, c):
            zdeg_v[pl.ds(i * 16, 16)] = z16
            return c

        lax.fori_loop(0, ROWS_PER_TILE // 16, zdeg, 0)

        for j in range(ROWS_PER_TILE // K):
            pltpu.async_copy(rows_a, acc_sh.at[pl.ds(r0 + j * K, K)], sem_a)
        pltpu.sync_copy(zdeg_v, deg_sh.at[pl.ds(r0, ROWS_PER_TILE)])
        for j in range(ROWS_PER_TILE // K):
            pltpu.make_async_copy(
                rows_a, acc_sh.at[pl.ds(r0, K)], sem_a).wait()

        for o in KOFF:
            ones_v[pl.ds(o, 16)] = jnp.full((16,), 1.0, jnp.float32)

        mask = jnp.full((16,), (1 << SHIFT) - 1, jnp.int32)

        def unpack(t, srcb, dstb):
            for o in KOFF:
                v = packed_v[pl.ds(t * K + o, 16)]
                srcb[pl.ds(o, 16)] = v & mask
                dstb[pl.ds(o, 16)] = lax.shift_right_logical(v, SHIFT)

        def gather(srcb, rows, sem):
            pltpu.async_copy(x_hbm.at[srcb], rows, sem)

        def gather_wait(rows, sem):
            pltpu.make_async_copy(x_hbm.at[srcb_a], rows, sem).wait()

        def deg_start(dstb):
            pltpu.async_copy(ones_v, deg_sh.at[dstb], sem_d, add=True)

        def deg_wait(dstb):
            pltpu.make_async_copy(ones_v, deg_sh.at[dstb], sem_d).wait()

        def scatter(dstb, rows):
            pltpu.sync_copy(rows, acc_sh.at[dstb], add=True)

        unpack(0, srcb_a, dstb_a)
        unpack(1, srcb_b, dstb_b)
        unpack(2, srcb_c, dstb_c)
        unpack(3, srcb_d, dstb_d)
        unpack(4, srcb_e, dstb_e)

        plsc.subcore_barrier()

        gather(srcb_a, rows_a, sem_a)
        gather(srcb_b, rows_b, sem_b)
        gather(srcb_c, rows_c, sem_c)
        gather(srcb_d, rows_d, sem_d2)
        gather(srcb_e, rows_e, sem_e)

        def chunk_step(tn, srcb, dstb, rows, sem):
            gather_wait(rows, sem)
            deg_start(dstb)
            scatter(dstb, rows)
            deg_wait(dstb)

            @pl.when(tn < CHUNKS)
            def _():
                unpack(tn, srcb, dstb)
                gather(srcb, rows, sem)

        def quint_body(i, c):
            t = 5 * i
            chunk_step(t + 5, srcb_a, dstb_a, rows_a, sem_a)
            chunk_step(t + 6, srcb_b, dstb_b, rows_b, sem_b)
            chunk_step(t + 7, srcb_c, dstb_c, rows_c, sem_c)
            chunk_step(t + 8, srcb_d, dstb_d, rows_d, sem_d2)
            chunk_step(t + 9, srcb_e, dstb_e, rows_e, sem_e)
            return c

        lax.fori_loop(0, CHUNKS // 5, quint_body, 0)

        plsc.subcore_barrier()

        pltpu.sync_copy(
            acc_sh.at[pl.ds(r0, ROWS_PER_TILE)],
            part_hbm.at[cid, pl.ds(r0, ROWS_PER_TILE)],
        )

        @pl.when(tid == 0)
        def _():
            pltpu.sync_copy(deg_sh, pdeg_hbm.at[cid])

    return k(x, packed)


def kernel(x, edge_index, W, b):
    ei = edge_index.astype(jnp.int32)
    packed = _pack(ei).reshape(N_EDGES)

    partials, pdeg = _sc_aggregate(x, packed)
    return _finalize(partials, pdeg, W.T, b.reshape(1, D))


# speedup vs baseline: 1.0527x; 1.0368x over previous
import functools

import jax
import jax.numpy as jnp
from jax import lax
from jax.experimental import pallas as pl
from jax.experimental.pallas import tpu as pltpu
from jax.experimental.pallas import tpu_sc as plsc

N_NODES = 10000
N_EDGES = 320000
D = 128

NC = 2
NS = 16
NW = NC * NS
E_PER_W = N_EDGES // NW
K = 40
CHUNKS = E_PER_W // K
KOFF = (0, 16, 24)
NP = 10240
ROWS_PER_TILE = NP // NS
SHIFT = 14
FBLK = 1280


def _pack(ei):
    def body(e_ref, o_ref):
        o_ref[...] = (e_ref[1] << SHIFT) | e_ref[0]

    return pl.pallas_call(
        body,
        out_shape=jax.ShapeDtypeStruct((N_EDGES,), jnp.int32),
    )(ei)


def _finalize(partials, pdeg, Wt, b2):
    def body(p_ref, d_ref, w_ref, b_ref, o_ref):
        s = p_ref[0] + p_ref[1]
        deg = d_ref[0] + d_ref[1]
        clip = jnp.maximum(deg, 1.0)
        sn = s / clip[:, None]
        scale = (deg / clip)[:, None]
        o_ref[...] = (
            jnp.dot(sn, w_ref[...], preferred_element_type=jnp.float32)
            + scale * b_ref[...]
        )

    return pl.pallas_call(
        body,
        grid=(NP // FBLK,),
        in_specs=[
            pl.BlockSpec((NC, FBLK, D), lambda i: (0, i, 0)),
            pl.BlockSpec((NC, FBLK), lambda i: (0, i)),
            pl.BlockSpec((D, D), lambda i: (0, 0)),
            pl.BlockSpec((1, D), lambda i: (0, 0)),
        ],
        out_specs=pl.BlockSpec((FBLK, D), lambda i: (i, 0)),
        out_shape=jax.ShapeDtypeStruct((N_NODES, D), jnp.float32),
    )(partials, pdeg, Wt, b2)


def _sc_aggregate(x, packed):
    mesh = plsc.VectorSubcoreMesh(core_axis_name="c", subcore_axis_name="s")

    @functools.partial(
        pl.kernel,
        mesh=mesh,
        out_type=[
            jax.ShapeDtypeStruct((NC, NP, D), jnp.float32),
            jax.ShapeDtypeStruct((NC, NP), jnp.float32),
        ],
        scratch_types=[
            pltpu.VMEM((E_PER_W,), jnp.int32),
            pltpu.VMEM((K,), jnp.int32),
            pltpu.VMEM((K,), jnp.int32),
            pltpu.VMEM((K,), jnp.int32),
            pltpu.VMEM((K,), jnp.int32),
            pltpu.VMEM((K,), jnp.int32),
            pltpu.VMEM((K,), jnp.int32),
            pltpu.VMEM((K,), jnp.int32),
            pltpu.VMEM((K,), jnp.int32),
            pltpu.VMEM((K,), jnp.int32),
            pltpu.VMEM((K,), jnp.int32),
            pltpu.VMEM((K, D), jnp.float32),
            pltpu.VMEM((K, D), jnp.float32),
            pltpu.VMEM((K, D), jnp.float32),
            pltpu.VMEM((K, D), jnp.float32),
            pltpu.VMEM((K, D), jnp.float32),
            pltpu.VMEM((K,), jnp.float32),
            pltpu.VMEM((ROWS_PER_TILE,), jnp.float32),
            pltpu.VMEM_SHARED((NP, D), jnp.float32),
            pltpu.VMEM_SHARED((NP,), jnp.float32),
            pltpu.SemaphoreType.DMA,
            pltpu.SemaphoreType.DMA,
            pltpu.SemaphoreType.DMA,
            pltpu.SemaphoreType.DMA,
            pltpu.SemaphoreType.DMA,
            pltpu.SemaphoreType.DMA,
        ],
    )
    def k(x_hbm, packed_hbm, part_hbm, pdeg_hbm,
          packed_v, srcb_a, srcb_b, srcb_c, srcb_d, srcb_e,
          dstb_a, dstb_b, dstb_c, dstb_d, dstb_e,
          rows_a, rows_b, rows_c, rows_d, rows_e, ones_v,
          zdeg_v, acc_sh, deg_sh,
          sem_a, sem_b, sem_c, sem_d2, sem_e, sem_d):
        cid = lax.axis_index("c")
        tid = lax.axis_index("s")
        wid = cid * NS + tid
        r0 = tid * ROWS_PER_TILE

        pltpu.sync_copy(
            packed_hbm.at[pl.ds(wid * E_PER_W, E_PER_W)], packed_v)

        z16 = jnp.zeros((16,), jnp.float32)

        def zrow(r, c):
            for j in range(8):
                rows_a[r, pl.ds(j * 16, 16)] = z16
            return c

        lax.fori_loop(0, K, zrow, 0)

        def zdeg(i, c):
            zdeg_v[pl.ds(i * 16, 16)] = z16
            return c

        lax.fori_loop(0, ROWS_PER_TILE // 16, zdeg, 0)

        for j in range(ROWS_PER_TILE // K):
            pltpu.async_copy(rows_a, acc_sh.at[pl.ds(r0 + j * K, K)], sem_a)
        pltpu.sync_copy(zdeg_v, deg_sh.at[pl.ds(r0, ROWS_PER_TILE)])
        for j in range(ROWS_PER_TILE // K):
            pltpu.make_async_copy(
                rows_a, acc_sh.at[pl.ds(r0, K)], sem_a).wait()

        for o in KOFF:
            ones_v[pl.ds(o, 16)] = jnp.full((16,), 1.0, jnp.float32)

        mask = jnp.full((16,), (1 << SHIFT) - 1, jnp.int32)

        def unpack(t, srcb, dstb):
            for o in KOFF:
                v = packed_v[pl.ds(t * K + o, 16)]
                srcb[pl.ds(o, 16)] = v & mask
                dstb[pl.ds(o, 16)] = lax.shift_right_logical(v, SHIFT)

        def gather(srcb, rows, sem):
            pltpu.async_copy(x_hbm.at[srcb], rows, sem)

        def gather_wait(rows, sem):
            pltpu.make_async_copy(x_hbm.at[srcb_a], rows, sem).wait()

        def deg_start(dstb):
            pltpu.async_copy(ones_v, deg_sh.at[dstb], sem_d, add=True)

        def deg_wait(dstb):
            pltpu.make_async_copy(ones_v, deg_sh.at[dstb], sem_d).wait()

        def scatter(dstb, rows):
            pltpu.sync_copy(rows, acc_sh.at[dstb], add=True)

        unpack(0, srcb_a, dstb_a)
        unpack(1, srcb_b, dstb_b)
        unpack(2, srcb_c, dstb_c)
        unpack(3, srcb_d, dstb_d)
        unpack(4, srcb_e, dstb_e)

        plsc.subcore_barrier()

        gather(srcb_a, rows_a, sem_a)
        gather(srcb_b, rows_b, sem_b)
        gather(srcb_c, rows_c, sem_c)
        gather(srcb_d, rows_d, sem_d2)
        gather(srcb_e, rows_e, sem_e)

        def chunk_step(tn, srcb, dstb, rows, sem):
            gather_wait(rows, sem)
            deg_start(dstb)
            scatter(dstb, rows)
            deg_wait(dstb)

            @pl.when(tn < CHUNKS)
            def _():
                unpack(tn, srcb, dstb)
                gather(srcb, rows, sem)

        def quint_body(i, c):
            t = 5 * i
            chunk_step(t + 5, srcb_a, dstb_a, rows_a, sem_a)
            chunk_step(t + 6, srcb_b, dstb_b, rows_b, sem_b)
            chunk_step(t + 7, srcb_c, dstb_c, rows_c, sem_c)
            chunk_step(t + 8, srcb_d, dstb_d, rows_d, sem_d2)
            chunk_step(t + 9, srcb_e, dstb_e, rows_e, sem_e)
            return c

        lax.fori_loop(0, CHUNKS // 5, quint_body, 0)

        plsc.subcore_barrier()

        pltpu.sync_copy(
            acc_sh.at[pl.ds(r0, ROWS_PER_TILE)],
            part_hbm.at[cid, pl.ds(r0, ROWS_PER_TILE)],
        )

        @pl.when(tid == 0)
        def _():
            pltpu.sync_copy(deg_sh, pdeg_hbm.at[cid])

    return k(x, packed)


def kernel(x, edge_index, W, b):
    ei = edge_index.astype(jnp.int32)
    packed = _pack(ei)

    partials, pdeg = _sc_aggregate(x, packed)
    return _finalize(partials, pdeg, W.T, b.reshape(1, D))
